# baseline (speedup 1.0000x reference)
import functools

import numpy as np
import jax
import jax.numpy as jnp
from jax import lax
from jax.experimental import pallas as pl
from jax.experimental.pallas import tpu as pltpu
from jax.experimental.pallas import tpu_sc as plsc


def _sinusoidal_pe_np(max_len, d_model):
    position = np.arange(max_len, dtype=np.float32)[:, None]
    div_term = np.exp(np.arange(0, d_model, 2, dtype=np.float32) * (-np.log(10000.0) / d_model))
    pe = np.zeros((max_len, d_model), dtype=np.float32)
    pe[:, 0::2] = np.sin(position * div_term)
    pe[:, 1::2] = np.cos(position * div_term)
    return pe


_PE = _sinusoidal_pe_np(512, 32)

_BT = 128


def _scores_body(pwcw_ref, pe_ref, x_ref, psig_ref, csig_ref, scores_ref):
    pw = jax.nn.sigmoid(pwcw_ref[0])
    cw = jax.nn.sigmoid(pwcw_ref[1])
    total = pw + cw
    pw = pw / total
    cw = cw / total
    address = jnp.concatenate([pw * pe_ref[...], cw * x_ref[...]], axis=1)
    sigs_t = jnp.concatenate(
        [jnp.sign(psig_ref[...]).T, jnp.sign(csig_ref[...]).T], axis=0)
    scores_ref[...] = jnp.dot(address, sigs_t,
                              preferred_element_type=jnp.float32)


def _run_scores(pwcw, pe_t, xf, pos_sig, content_sig):
    T, _ = xf.shape
    E = pos_sig.shape[0]
    return pl.pallas_call(
        _scores_body,
        in_specs=[
            pl.BlockSpec(memory_space=pltpu.SMEM),
            pl.BlockSpec(memory_space=pltpu.VMEM),
            pl.BlockSpec(memory_space=pltpu.VMEM),
            pl.BlockSpec(memory_space=pltpu.VMEM),
            pl.BlockSpec(memory_space=pltpu.VMEM),
        ],
        out_shape=jax.ShapeDtypeStruct((T, E), jnp.float32),
    )(pwcw, pe_t, xf, pos_sig, content_sig)


def _plan_body(idx_in_ref, dest_ref, offs_ref, pi_ref, pe_ref, vv_ref, ff_ref):
    T = idx_in_ref.shape[0]
    E = 8
    idx = idx_in_ref[...]
    lane = lax.broadcasted_iota(jnp.int32, (T, E), 1)
    onehot = (lane == idx).astype(jnp.float32)
    bk = 128
    nb = T // bk
    prefix = []
    running = jnp.zeros((1, E), jnp.float32)
    for b in range(nb):
        prefix.append(running)
        running = running + jnp.sum(
            onehot[b * bk:(b + 1) * bk, :], axis=0, keepdims=True)
    counts = running
    jj = lax.broadcasted_iota(jnp.int32, (16, E), 0)
    kk = lax.broadcasted_iota(jnp.int32, (16, E), 1)
    cb16 = jnp.broadcast_to(counts, (16, E))
    offs_col = jnp.sum(jnp.where(kk < jj, cb16, 0.0), axis=1, keepdims=True)
    offs_ref[...] = offs_col.astype(jnp.int32)
    cbT = jnp.broadcast_to(counts, (T, E))
    base = jnp.sum(jnp.where(lane < idx, cbT, 0.0), axis=1, keepdims=True)
    rr = lax.broadcasted_iota(jnp.int32, (bk, bk), 0)
    cc = lax.broadcasted_iota(jnp.int32, (bk, bk), 1)
    l128 = (cc <= rr).astype(jnp.float32)
    for b in range(nb):
        oh_b = onehot[b * bk:(b + 1) * bk, :]
        csum_b = jnp.dot(l128, oh_b, preferred_element_type=jnp.float32) + prefix[b]
        rank_b = jnp.sum((csum_b - 1.0) * oh_b, axis=1, keepdims=True)
        dest_ref[b * bk:(b + 1) * bk, :] = (
            base[b * bk:(b + 1) * bk, :] + rank_b).astype(jnp.int32)

    nbf = T // _BT
    offs_f = jnp.broadcast_to(offs_col, (16, 32))
    sub16 = lax.broadcasted_iota(jnp.int32, (16, 32), 0)
    lane32 = lax.broadcasted_iota(jnp.int32, (16, 32), 1)
    jmask = (sub16 >= 1) & (sub16 <= E)
    lo_row = (lane32 * _BT).astype(jnp.float32)
    hi_row = lo_row + np.float32(_BT)
    emin_row = jnp.sum(jnp.where(jmask & (offs_f <= lo_row), 1.0, 0.0),
                       axis=0, keepdims=True)
    emax_row = jnp.sum(jnp.where(jmask & (offs_f < hi_row), 1.0, 0.0),
                       axis=0, keepdims=True)
    iv_row = (lane32[:1] < nbf)
    num_row = jnp.where(iv_row, emax_row - emin_row + 1.0, 0.0)

    def _to_col(row):
        return jnp.sum(jnp.where(sub16 == lane32,
                                 jnp.broadcast_to(row, (16, 32)), 0.0),
                       axis=1, keepdims=True)

    num_col = _to_col(num_row)
    emin_col = _to_col(emin_row)
    emax_col = _to_col(emax_row)
    num_b = jnp.broadcast_to(num_col, (16, 32))
    ss_row = jnp.sum(jnp.where(sub16 < lane32, num_b, 0.0),
                     axis=0, keepdims=True)
    ss_col = _to_col(ss_row)
    ss_b = jnp.broadcast_to(ss_col, (16, 32))
    s_lane = lane32[:1].astype(jnp.float32)
    ivalid = sub16 < nbf
    pair_i = jnp.sum(jnp.where(ivalid & (ss_b <= s_lane), 1.0, 0.0),
                     axis=0, keepdims=True) - 1.0
    pim = sub16 == pair_i.astype(jnp.int32)
    emin_at = jnp.sum(jnp.where(pim, jnp.broadcast_to(emin_col, (16, 32)), 0.0),
                      axis=0, keepdims=True)
    emax_at = jnp.sum(jnp.where(pim, jnp.broadcast_to(emax_col, (16, 32)), 0.0),
                      axis=0, keepdims=True)
    ss_at = jnp.sum(jnp.where(pim, ss_b, 0.0), axis=0, keepdims=True)
    pair_e = jnp.minimum(emin_at + (s_lane - ss_at), emax_at)
    total = jnp.sum(num_b[:, :1], axis=0, keepdims=True)
    valid = (s_lane < jnp.broadcast_to(total, (1, 32))).astype(jnp.int32)
    first = (s_lane == ss_at).astype(jnp.int32)
    pi_ref[...] = pair_i.astype(jnp.int32)
    pe_ref[...] = pair_e.astype(jnp.int32)
    vv_ref[...] = valid
    ff_ref[...] = first


def _run_plan(idx):
    T = idx.shape[0]
    return pl.pallas_call(
        _plan_body,
        in_specs=[pl.BlockSpec(memory_space=pltpu.VMEM)],
        out_shape=[
            jax.ShapeDtypeStruct((T, 1), jnp.int32),
            jax.ShapeDtypeStruct((16, 1), jnp.int32),
            jax.ShapeDtypeStruct((1, 32), jnp.int32),
            jax.ShapeDtypeStruct((1, 32), jnp.int32),
            jax.ShapeDtypeStruct((1, 32), jnp.int32),
            jax.ShapeDtypeStruct((1, 32), jnp.int32),
        ],
    )(idx)


def _ffn_body(offs_ref, pi_ref, pe_ref, vv_ref, ff_ref,
              x_ref, w1_ref, b1_ref, w2_ref, b2_ref, out_ref):
    s = pl.program_id(0)
    e = pe_ref[s]
    lo = pi_ref[s] * _BT
    start = offs_ref[e]
    end = offs_ref[e + 1]

    @pl.when(vv_ref[s] == 1)
    def _compute():
        xb = x_ref[...].astype(jnp.bfloat16)
        h = jnp.dot(xb, w1_ref[0].astype(jnp.bfloat16),
                    preferred_element_type=jnp.float32) + b1_ref[0]
        h = h * 0.5 * (1.0 + lax.erf(h * np.float32(0.7071067811865476)))
        y = jnp.dot(h.astype(jnp.bfloat16), w2_ref[0].astype(jnp.bfloat16),
                    preferred_element_type=jnp.float32) + b2_ref[0]
        rows = lo + lax.broadcasted_iota(jnp.int32, (_BT, 1), 0)
        m = (rows >= start) & (rows < end)
        yy = jnp.where(m, y, 0.0)

        @pl.when(ff_ref[s] == 1)
        def _first():
            out_ref[...] = yy

        @pl.when(ff_ref[s] == 0)
        def _accum():
            out_ref[...] += yy


def _run_ffn(x_sorted, W1, b1, W2, b2, offs16, pair_i, pair_e, valid, first):
    T, D = x_sorted.shape
    E, _, H = W1.shape
    nslot = T // _BT + E - 1
    grid_spec = pltpu.PrefetchScalarGridSpec(
        num_scalar_prefetch=5,
        grid=(nslot,),
        in_specs=[
            pl.BlockSpec((_BT, D), lambda s, offs, pi, pe, vv, ff: (pi[s], 0)),
            pl.BlockSpec((1, D, H), lambda s, offs, pi, pe, vv, ff: (pe[s], 0, 0)),
            pl.BlockSpec((1, 1, H), lambda s, offs, pi, pe, vv, ff: (pe[s], 0, 0)),
            pl.BlockSpec((1, H, D), lambda s, offs, pi, pe, vv, ff: (pe[s], 0, 0)),
            pl.BlockSpec((1, 1, D), lambda s, offs, pi, pe, vv, ff: (pe[s], 0, 0)),
        ],
        out_specs=pl.BlockSpec((_BT, D), lambda s, offs, pi, pe, vv, ff: (pi[s], 0)),
    )
    return pl.pallas_call(
        _ffn_body,
        grid_spec=grid_spec,
        out_shape=jax.ShapeDtypeStruct((T, D), jnp.float32),
        compiler_params=pltpu.CompilerParams(
            dimension_semantics=("arbitrary",)),
    )(offs16, pair_i, pair_e, valid, first,
      x_sorted, W1, b1.reshape(E, 1, H), W2, b2.reshape(E, 1, D))


def _sc_permute(src, dest_idx, direction):
    T, D = src.shape
    info = plsc.get_sparse_core_info()
    nc, ns = info.num_cores, info.num_subcores
    nw = nc * ns
    bpw = T // nw
    mesh = plsc.VectorSubcoreMesh(core_axis_name="c", subcore_axis_name="s")

    @functools.partial(
        pl.kernel,
        mesh=mesh,
        out_type=jax.ShapeDtypeStruct((T, D), jnp.float32),
        scratch_types=[
            pltpu.VMEM((bpw,), jnp.int32),
            pltpu.VMEM((bpw, D), jnp.float32),
            pltpu.SemaphoreType.DMA,
        ],
    )
    def k(src_hbm, dest_hbm, out_hbm, idx_v, rows_v, sem):
        wid = lax.axis_index("s") * nc + lax.axis_index("c")
        base = wid * bpw
        pltpu.sync_copy(dest_hbm.at[pl.ds(base, bpw)], idx_v)
        if direction == "scatter":
            pltpu.sync_copy(src_hbm.at[pl.ds(base, bpw)], rows_v)
            pltpu.async_copy(rows_v, out_hbm.at[idx_v], sem).wait()
        else:
            pltpu.async_copy(src_hbm.at[idx_v], rows_v, sem).wait()
            pltpu.sync_copy(rows_v, out_hbm.at[pl.ds(base, bpw)])

    return k(src, dest_idx)


def kernel(x, position_weight, content_weight, pos_sig, content_sig, W1, b1, W2, b2):
    B, S, D = x.shape
    T = B * S
    E, _, H = W1.shape
    xf = x.reshape(T, D)
    pe = jnp.asarray(_PE)[:S]
    pe_t = jnp.broadcast_to(pe[None, :, :], (B, S, _PE.shape[1])).reshape(T, -1)
    pwcw = jnp.stack([position_weight, content_weight])

    scores = _run_scores(pwcw, pe_t, xf, pos_sig, content_sig)

    fast_idx = jnp.argmax(scores, axis=-1).astype(jnp.int32)
    m1 = jnp.max(scores, axis=-1)
    masked = jnp.where(
        jax.nn.one_hot(fast_idx, E, dtype=jnp.bool_), -jnp.inf, scores)
    gap_min = jnp.min(m1 - jnp.max(masked, axis=-1))

    def _exact_decision(_):
        pos_enc = jnp.broadcast_to(pe[None, :, :], (B, S, _PE.shape[1]))
        pw = jax.nn.sigmoid(position_weight)
        cw = jax.nn.sigmoid(content_weight)
        total = pw + cw
        pw = pw / total
        cw = cw / total
        address = jnp.concatenate([pw * pos_enc, cw * x], axis=-1)
        signatures = jnp.concatenate(
            [jnp.sign(pos_sig), jnp.sign(content_sig)], axis=-1)
        decision_scores = jnp.einsum('bsd,td->bst', address, signatures)
        return jnp.argmax(decision_scores, axis=-1).reshape(T).astype(jnp.int32)

    indices = lax.cond(gap_min < 5e-4, _exact_decision,
                       lambda _: fast_idx, operand=None)

    dest, offs16, pair_i, pair_e, valid, first = _run_plan(indices.reshape(T, 1))
    dest_flat = dest.reshape(T)
    x_sorted = _sc_permute(xf, dest_flat, "scatter")
    y_sorted = _run_ffn(x_sorted, W1, b1, W2, b2, offs16.reshape(16),
                        pair_i.reshape(32), pair_e.reshape(32),
                        valid.reshape(32), first.reshape(32))
    outf = _sc_permute(y_sorted, dest_flat, "gather")
    return outf.reshape(B, S, D), indices.reshape(B, S), scores.reshape(B, S, E)


# speedup vs baseline: 1.0627x; 1.0627x over previous
import functools

import numpy as np
import jax
import jax.numpy as jnp
from jax import lax
from jax.experimental import pallas as pl
from jax.experimental.pallas import tpu as pltpu
from jax.experimental.pallas import tpu_sc as plsc


def _sinusoidal_pe_np(max_len, d_model):
    position = np.arange(max_len, dtype=np.float32)[:, None]
    div_term = np.exp(np.arange(0, d_model, 2, dtype=np.float32) * (-np.log(10000.0) / d_model))
    pe = np.zeros((max_len, d_model), dtype=np.float32)
    pe[:, 0::2] = np.sin(position * div_term)
    pe[:, 1::2] = np.cos(position * div_term)
    return pe


_PE = _sinusoidal_pe_np(512, 32)

_BT = 256


def _scores_body(pwcw_ref, pe_ref, x_ref, psig_ref, csig_ref, scores_ref):
    pw = jax.nn.sigmoid(pwcw_ref[0])
    cw = jax.nn.sigmoid(pwcw_ref[1])
    total = pw + cw
    pw = pw / total
    cw = cw / total
    address = jnp.concatenate([pw * pe_ref[...], cw * x_ref[...]], axis=1)
    sigs_t = jnp.concatenate(
        [jnp.sign(psig_ref[...]).T, jnp.sign(csig_ref[...]).T], axis=0)
    scores_ref[...] = jnp.dot(address, sigs_t,
                              preferred_element_type=jnp.float32)


def _run_scores(pwcw, pe_t, xf, pos_sig, content_sig):
    T, _ = xf.shape
    E = pos_sig.shape[0]
    return pl.pallas_call(
        _scores_body,
        in_specs=[
            pl.BlockSpec(memory_space=pltpu.SMEM),
            pl.BlockSpec(memory_space=pltpu.VMEM),
            pl.BlockSpec(memory_space=pltpu.VMEM),
            pl.BlockSpec(memory_space=pltpu.VMEM),
            pl.BlockSpec(memory_space=pltpu.VMEM),
        ],
        out_shape=jax.ShapeDtypeStruct((T, E), jnp.float32),
    )(pwcw, pe_t, xf, pos_sig, content_sig)


def _plan_body(idx_in_ref, dest_ref, offs_ref, pi_ref, pe_ref, vv_ref, ff_ref):
    T = idx_in_ref.shape[0]
    E = 8
    idx = idx_in_ref[...]
    lane = lax.broadcasted_iota(jnp.int32, (T, E), 1)
    onehot = (lane == idx).astype(jnp.float32)
    bk = 128
    nb = T // bk
    prefix = []
    running = jnp.zeros((1, E), jnp.float32)
    for b in range(nb):
        prefix.append(running)
        running = running + jnp.sum(
            onehot[b * bk:(b + 1) * bk, :], axis=0, keepdims=True)
    counts = running
    jj = lax.broadcasted_iota(jnp.int32, (16, E), 0)
    kk = lax.broadcasted_iota(jnp.int32, (16, E), 1)
    cb16 = jnp.broadcast_to(counts, (16, E))
    offs_col = jnp.sum(jnp.where(kk < jj, cb16, 0.0), axis=1, keepdims=True)
    offs_ref[...] = offs_col.astype(jnp.int32)
    cbT = jnp.broadcast_to(counts, (T, E))
    base = jnp.sum(jnp.where(lane < idx, cbT, 0.0), axis=1, keepdims=True)
    rr = lax.broadcasted_iota(jnp.int32, (bk, bk), 0)
    cc = lax.broadcasted_iota(jnp.int32, (bk, bk), 1)
    l128 = (cc <= rr).astype(jnp.float32)
    for b in range(nb):
        oh_b = onehot[b * bk:(b + 1) * bk, :]
        csum_b = jnp.dot(l128, oh_b, preferred_element_type=jnp.float32) + prefix[b]
        rank_b = jnp.sum((csum_b - 1.0) * oh_b, axis=1, keepdims=True)
        dest_ref[b * bk:(b + 1) * bk, :] = (
            base[b * bk:(b + 1) * bk, :] + rank_b).astype(jnp.int32)

    nbf = T // _BT
    offs_f = jnp.broadcast_to(offs_col, (16, 32))
    sub16 = lax.broadcasted_iota(jnp.int32, (16, 32), 0)
    lane32 = lax.broadcasted_iota(jnp.int32, (16, 32), 1)
    jmask = (sub16 >= 1) & (sub16 <= E)
    lo_row = (lane32 * _BT).astype(jnp.float32)
    hi_row = lo_row + np.float32(_BT)
    emin_row = jnp.sum(jnp.where(jmask & (offs_f <= lo_row), 1.0, 0.0),
                       axis=0, keepdims=True)
    emax_row = jnp.sum(jnp.where(jmask & (offs_f < hi_row), 1.0, 0.0),
                       axis=0, keepdims=True)
    iv_row = (lane32[:1] < nbf)
    num_row = jnp.where(iv_row, emax_row - emin_row + 1.0, 0.0)

    def _to_col(row):
        return jnp.sum(jnp.where(sub16 == lane32,
                                 jnp.broadcast_to(row, (16, 32)), 0.0),
                       axis=1, keepdims=True)

    num_col = _to_col(num_row)
    emin_col = _to_col(emin_row)
    emax_col = _to_col(emax_row)
    num_b = jnp.broadcast_to(num_col, (16, 32))
    ss_row = jnp.sum(jnp.where(sub16 < lane32, num_b, 0.0),
                     axis=0, keepdims=True)
    ss_col = _to_col(ss_row)
    ss_b = jnp.broadcast_to(ss_col, (16, 32))
    s_lane = lane32[:1].astype(jnp.float32)
    ivalid = sub16 < nbf
    pair_i = jnp.sum(jnp.where(ivalid & (ss_b <= s_lane), 1.0, 0.0),
                     axis=0, keepdims=True) - 1.0
    pim = sub16 == pair_i.astype(jnp.int32)
    emin_at = jnp.sum(jnp.where(pim, jnp.broadcast_to(emin_col, (16, 32)), 0.0),
                      axis=0, keepdims=True)
    emax_at = jnp.sum(jnp.where(pim, jnp.broadcast_to(emax_col, (16, 32)), 0.0),
                      axis=0, keepdims=True)
    ss_at = jnp.sum(jnp.where(pim, ss_b, 0.0), axis=0, keepdims=True)
    pair_e = jnp.minimum(emin_at + (s_lane - ss_at), emax_at)
    total = jnp.sum(num_b[:, :1], axis=0, keepdims=True)
    valid = (s_lane < jnp.broadcast_to(total, (1, 32))).astype(jnp.int32)
    first = (s_lane == ss_at).astype(jnp.int32)
    pi_ref[...] = pair_i.astype(jnp.int32)
    pe_ref[...] = pair_e.astype(jnp.int32)
    vv_ref[...] = valid
    ff_ref[...] = first


def _run_plan(idx):
    T = idx.shape[0]
    return pl.pallas_call(
        _plan_body,
        in_specs=[pl.BlockSpec(memory_space=pltpu.VMEM)],
        out_shape=[
            jax.ShapeDtypeStruct((T, 1), jnp.int32),
            jax.ShapeDtypeStruct((16, 1), jnp.int32),
            jax.ShapeDtypeStruct((1, 32), jnp.int32),
            jax.ShapeDtypeStruct((1, 32), jnp.int32),
            jax.ShapeDtypeStruct((1, 32), jnp.int32),
            jax.ShapeDtypeStruct((1, 32), jnp.int32),
        ],
    )(idx)


def _ffn_body(offs_ref, pi_ref, pe_ref, vv_ref, ff_ref,
              x_ref, w1_ref, b1_ref, w2_ref, b2_ref, out_ref):
    s = pl.program_id(0)
    e = pe_ref[s]
    lo = pi_ref[s] * _BT
    start = offs_ref[e]
    end = offs_ref[e + 1]

    @pl.when(vv_ref[s] == 1)
    def _compute():
        xb = x_ref[...].astype(jnp.bfloat16)
        h = jnp.dot(xb, w1_ref[0].astype(jnp.bfloat16),
                    preferred_element_type=jnp.float32) + b1_ref[0]
        h = h * 0.5 * (1.0 + lax.erf(h * np.float32(0.7071067811865476)))
        y = jnp.dot(h.astype(jnp.bfloat16), w2_ref[0].astype(jnp.bfloat16),
                    preferred_element_type=jnp.float32) + b2_ref[0]
        rows = lo + lax.broadcasted_iota(jnp.int32, (_BT, 1), 0)
        m = (rows >= start) & (rows < end)
        yy = jnp.where(m, y, 0.0)

        @pl.when(ff_ref[s] == 1)
        def _first():
            out_ref[...] = yy

        @pl.when(ff_ref[s] == 0)
        def _accum():
            out_ref[...] += yy


def _run_ffn(x_sorted, W1, b1, W2, b2, offs16, pair_i, pair_e, valid, first):
    T, D = x_sorted.shape
    E, _, H = W1.shape
    nslot = T // _BT + E - 1
    grid_spec = pltpu.PrefetchScalarGridSpec(
        num_scalar_prefetch=5,
        grid=(nslot,),
        in_specs=[
            pl.BlockSpec((_BT, D), lambda s, offs, pi, pe, vv, ff: (pi[s], 0)),
            pl.BlockSpec((1, D, H), lambda s, offs, pi, pe, vv, ff: (pe[s], 0, 0)),
            pl.BlockSpec((1, 1, H), lambda s, offs, pi, pe, vv, ff: (pe[s], 0, 0)),
            pl.BlockSpec((1, H, D), lambda s, offs, pi, pe, vv, ff: (pe[s], 0, 0)),
            pl.BlockSpec((1, 1, D), lambda s, offs, pi, pe, vv, ff: (pe[s], 0, 0)),
        ],
        out_specs=pl.BlockSpec((_BT, D), lambda s, offs, pi, pe, vv, ff: (pi[s], 0)),
    )
    return pl.pallas_call(
        _ffn_body,
        grid_spec=grid_spec,
        out_shape=jax.ShapeDtypeStruct((T, D), jnp.float32),
        compiler_params=pltpu.CompilerParams(
            dimension_semantics=("arbitrary",)),
    )(offs16, pair_i, pair_e, valid, first,
      x_sorted, W1, b1.reshape(E, 1, H), W2, b2.reshape(E, 1, D))


def _sc_permute(src, dest_idx, direction):
    T, D = src.shape
    info = plsc.get_sparse_core_info()
    nc, ns = info.num_cores, info.num_subcores
    nw = nc * ns
    bpw = T // nw
    mesh = plsc.VectorSubcoreMesh(core_axis_name="c", subcore_axis_name="s")

    @functools.partial(
        pl.kernel,
        mesh=mesh,
        out_type=jax.ShapeDtypeStruct((T, D), jnp.float32),
        scratch_types=[
            pltpu.VMEM((bpw,), jnp.int32),
            pltpu.VMEM((bpw, D), jnp.float32),
            pltpu.SemaphoreType.DMA,
        ],
    )
    def k(src_hbm, dest_hbm, out_hbm, idx_v, rows_v, sem):
        wid = lax.axis_index("s") * nc + lax.axis_index("c")
        base = wid * bpw
        pltpu.sync_copy(dest_hbm.at[pl.ds(base, bpw)], idx_v)
        if direction == "scatter":
            pltpu.sync_copy(src_hbm.at[pl.ds(base, bpw)], rows_v)
            pltpu.async_copy(rows_v, out_hbm.at[idx_v], sem).wait()
        else:
            pltpu.async_copy(src_hbm.at[idx_v], rows_v, sem).wait()
            pltpu.sync_copy(rows_v, out_hbm.at[pl.ds(base, bpw)])

    return k(src, dest_idx)


def kernel(x, position_weight, content_weight, pos_sig, content_sig, W1, b1, W2, b2):
    B, S, D = x.shape
    T = B * S
    E, _, H = W1.shape
    xf = x.reshape(T, D)
    pe = jnp.asarray(_PE)[:S]
    pe_t = jnp.broadcast_to(pe[None, :, :], (B, S, _PE.shape[1])).reshape(T, -1)
    pwcw = jnp.stack([position_weight, content_weight])

    scores = _run_scores(pwcw, pe_t, xf, pos_sig, content_sig)

    fast_idx = jnp.argmax(scores, axis=-1).astype(jnp.int32)
    m1 = jnp.max(scores, axis=-1)
    masked = jnp.where(
        jax.nn.one_hot(fast_idx, E, dtype=jnp.bool_), -jnp.inf, scores)
    gap_min = jnp.min(m1 - jnp.max(masked, axis=-1))

    def _exact_decision(_):
        pos_enc = jnp.broadcast_to(pe[None, :, :], (B, S, _PE.shape[1]))
        pw = jax.nn.sigmoid(position_weight)
        cw = jax.nn.sigmoid(content_weight)
        total = pw + cw
        pw = pw / total
        cw = cw / total
        address = jnp.concatenate([pw * pos_enc, cw * x], axis=-1)
        signatures = jnp.concatenate(
            [jnp.sign(pos_sig), jnp.sign(content_sig)], axis=-1)
        decision_scores = jnp.einsum('bsd,td->bst', address, signatures)
        return jnp.argmax(decision_scores, axis=-1).reshape(T).astype(jnp.int32)

    indices = lax.cond(gap_min < 5e-4, _exact_decision,
                       lambda _: fast_idx, operand=None)

    dest, offs16, pair_i, pair_e, valid, first = _run_plan(indices.reshape(T, 1))
    dest_flat = dest.reshape(T)
    x_sorted = _sc_permute(xf, dest_flat, "scatter")
    y_sorted = _run_ffn(x_sorted, W1, b1, W2, b2, offs16.reshape(16),
                        pair_i.reshape(32), pair_e.reshape(32),
                        valid.reshape(32), first.reshape(32))
    outf = _sc_permute(y_sorted, dest_flat, "gather")
    return outf.reshape(B, S, D), indices.reshape(B, S), scores.reshape(B, S, E)
